# Initial kernel scaffold; baseline (speedup 1.0000x reference)
import jax
import jax.numpy as jnp
from jax.experimental import pallas as pl


def kernel(x, edge_index, tails, gen_feat, num_pred):
    raise NotImplementedError("write your pallas kernel here")


# speedup vs baseline: 2.2925x; 2.2925x over previous
import jax
import jax.numpy as jnp
from jax import lax
from jax.experimental import pallas as pl
from jax.experimental.pallas import tpu as pltpu
from jax.experimental.pallas import tpu_sc as plsc


def kernel(x, edge_index, tails, gen_feat, num_pred):
    N, D = x.shape
    E = edge_index.shape[1]
    T = tails.shape[0]
    P = gen_feat.shape[0] // T
    G = T * P
    ND = N * D
    GD = gen_feat.shape[0] * D
    W = E + G

    info = plsc.get_sparse_core_info()
    NC, NS = info.num_cores, info.num_subcores
    NW = NC * NS

    FX = ND // NW
    FG = GD // NW
    EC = E // NW
    GC = max(16, G // NW)
    while G % GC or GC % 16:
        GC += 1
    NACT = G // GC

    mesh = plsc.VectorSubcoreMesh(core_axis_name="c", subcore_axis_name="s")

    def body(x_h, gen_h, edge_h, tails_h, feat_o, edge_o,
             bufa, bufb, eb0, eb1, tails_v, rep_v, ids_v, s0, s1, s2, s3):
        wid = lax.axis_index("s") * NC + lax.axis_index("c")

        da = pltpu.async_copy(x_h.at[pl.ds(wid * FX, FX)], bufa, s0)
        db = pltpu.async_copy(gen_h.at[pl.ds(wid * FG, FG)], bufb, s1)
        d0 = pltpu.async_copy(edge_h.at[pl.ds(wid * EC, EC)], eb0, s2)
        d1 = pltpu.async_copy(edge_h.at[pl.ds(E + wid * EC, EC)], eb1, s3)

        @pl.when(wid < NACT)
        def _gen():
            pltpu.sync_copy(tails_h, tails_v)
            c0 = wid * GC
            iota = lax.iota(jnp.int32, 16)
            for j in range(GC // 16):
                k = iota + (c0 + j * 16)
                rep_v[pl.ds(j * 16, 16)] = plsc.load_gather(tails_v, [k // P])
                ids_v[pl.ds(j * 16, 16)] = k + N
            pltpu.sync_copy(rep_v, edge_o.at[pl.ds(E + c0, GC)])
            pltpu.sync_copy(ids_v, edge_o.at[pl.ds(W + E + c0, GC)])

        da.wait()
        oa = pltpu.async_copy(bufa, feat_o.at[pl.ds(wid * FX, FX)], s0)
        db.wait()
        ob = pltpu.async_copy(bufb, feat_o.at[pl.ds(ND + wid * FG, FG)], s1)
        d0.wait()
        o0 = pltpu.async_copy(eb0, edge_o.at[pl.ds(wid * EC, EC)], s2)
        d1.wait()
        o1 = pltpu.async_copy(eb1, edge_o.at[pl.ds(W + wid * EC, EC)], s3)
        oa.wait()
        ob.wait()
        o0.wait()
        o1.wait()

    run = pl.kernel(
        body,
        out_type=[
            jax.ShapeDtypeStruct((ND + GD,), jnp.float32),
            jax.ShapeDtypeStruct((2 * W,), jnp.int32),
        ],
        mesh=mesh,
        scratch_types=[
            pltpu.VMEM((FX,), jnp.float32),
            pltpu.VMEM((FG,), jnp.float32),
            pltpu.VMEM((EC,), jnp.int32),
            pltpu.VMEM((EC,), jnp.int32),
            pltpu.VMEM((T,), jnp.int32),
            pltpu.VMEM((GC,), jnp.int32),
            pltpu.VMEM((GC,), jnp.int32),
            pltpu.SemaphoreType.DMA,
            pltpu.SemaphoreType.DMA,
            pltpu.SemaphoreType.DMA,
            pltpu.SemaphoreType.DMA,
        ],
        compiler_params=pltpu.CompilerParams(needs_layout_passes=False),
    )

    feat_flat, edge_flat = run(
        x.reshape(-1),
        gen_feat.astype(jnp.float32).reshape(-1),
        edge_index.reshape(-1),
        tails,
    )
    return (feat_flat.reshape(N + G, D), edge_flat.reshape(2, W))
